# baseline (speedup 1.0000x reference)
import functools

import jax
import jax.numpy as jnp
from jax import lax
from jax.experimental import pallas as pl
from jax.experimental.pallas import tpu as pltpu
from jax.experimental.pallas import tpu_sc as plsc

_BM = 2048
_BN = 2048
_K = 256
_N = 8192


def _vq_argmin_body(z_ref, z2_ref, e_ref, idx_ref, e2_ref, eb_ref):
    i = pl.program_id(0)

    @pl.when(i == 0)
    def _():
        e_all = e_ref[...]
        e2_ref[...] = jnp.sum(e_all * e_all, axis=1, keepdims=True)
        eb_ref[...] = e_all.astype(jnp.bfloat16)

    z = z_ref[...]
    z2t = z2_ref[...]
    zb2 = z.astype(jnp.bfloat16) * jnp.bfloat16(2.0)
    sub8 = lax.broadcasted_iota(jnp.int32, (8, _BM), 0)
    best_val = jnp.full((1, _BM), jnp.inf, dtype=jnp.float32)
    best_idx = jnp.zeros((1, _BM), dtype=jnp.int32)
    for j in range(_N // _BN):
        eb = eb_ref[j * _BN:(j + 1) * _BN, :]
        dot2 = lax.dot_general(eb, zb2, (((1,), (1,)), ((), ())),
                               preferred_element_type=jnp.float32)
        acc_v = jnp.full((8, _BM), jnp.inf, dtype=jnp.float32)
        acc_t = jnp.zeros((8, _BM), dtype=jnp.int32)
        for t in range(_BN // 8):
            r0 = j * _BN + t * 8
            d = (z2t + e2_ref[r0:r0 + 8, :]) - dot2[t * 8:(t + 1) * 8, :]
            m = d < acc_v
            acc_v = jnp.where(m, d, acc_v)
            acc_t = jnp.where(m, jnp.int32(t), acc_t)
        row_min = jnp.min(acc_v, axis=0, keepdims=True)
        n_val = acc_t * 8 + sub8 + j * _BN
        in_idx = jnp.min(jnp.where(acc_v == row_min, n_val, _N),
                         axis=0, keepdims=True)
        upd = row_min < best_val
        best_val = jnp.where(
            upd, row_min.astype(jnp.bfloat16).astype(jnp.float32), best_val)
        best_idx = jnp.where(upd, in_idx, best_idx)
    idx_ref[...] = best_idx


def _argmin_call(z_flat, z2t, embedding):
    b = z_flat.shape[0]
    return pl.pallas_call(
        _vq_argmin_body,
        grid=(b // _BM,),
        in_specs=[
            pl.BlockSpec((_BM, _K), lambda i: (i, 0)),
            pl.BlockSpec((1, _BM), lambda i: (0, i)),
            pl.BlockSpec((_N, _K), lambda i: (0, 0)),
        ],
        out_specs=pl.BlockSpec((1, _BM), lambda i: (0, i)),
        out_shape=jax.ShapeDtypeStruct((1, b), jnp.int32),
        scratch_shapes=[pltpu.VMEM((_N, 1), jnp.float32),
                        pltpu.VMEM((_N, _K), jnp.bfloat16)],
    )(z_flat, z2t, embedding)


def _gather_call(embedding, idx_flat):
    b = idx_flat.shape[0]
    d = embedding.shape[1]
    info = plsc.get_sparse_core_info()
    nc, ns = info.num_cores, info.num_subcores
    nw = nc * ns
    bpw = b // nw

    mesh = plsc.VectorSubcoreMesh(core_axis_name="c", subcore_axis_name="s")

    h = bpw // 2

    @functools.partial(
        pl.kernel, mesh=mesh,
        out_type=jax.ShapeDtypeStruct((b, d), jnp.float32),
        scratch_types=[
            pltpu.VMEM((bpw,), jnp.int32),
            pltpu.VMEM((bpw, d), jnp.float32),
            pltpu.SemaphoreType.DMA,
            pltpu.SemaphoreType.DMA,
        ],
    )
    def _gather_rows(table_hbm, idx_hbm, out_hbm, idx_v, rows_v, sem_g, sem_w):
        wid = lax.axis_index("s") * nc + lax.axis_index("c")
        base = wid * bpw
        pltpu.sync_copy(idx_hbm.at[pl.ds(base, bpw)], idx_v)
        g0 = pltpu.async_copy(table_hbm.at[idx_v.at[pl.ds(0, h)]],
                              rows_v.at[pl.ds(0, h)], sem_g)
        g1 = pltpu.async_copy(table_hbm.at[idx_v.at[pl.ds(h, h)]],
                              rows_v.at[pl.ds(h, h)], sem_g)
        g0.wait()
        w0 = pltpu.async_copy(rows_v.at[pl.ds(0, h)],
                              out_hbm.at[pl.ds(base, h)], sem_w)
        g1.wait()
        w1 = pltpu.async_copy(rows_v.at[pl.ds(h, h)],
                              out_hbm.at[pl.ds(base + h, h)], sem_w)
        w0.wait()
        w1.wait()

    return _gather_rows(embedding, idx_flat)


def kernel(z, embedding, is_training):
    e_dim = embedding.shape[1]
    z_flat = z.reshape(-1, e_dim)
    z2t = jnp.sum(z ** 2, axis=-1).reshape(1, -1)
    idx = _argmin_call(z_flat, z2t, embedding).reshape(-1)
    z_q = _gather_call(embedding, idx).reshape(z.shape)
    return (z_q, idx)


# speedup vs baseline: 1.0076x; 1.0076x over previous
import functools

import jax
import jax.numpy as jnp
from jax import lax
from jax.experimental import pallas as pl
from jax.experimental.pallas import tpu as pltpu
from jax.experimental.pallas import tpu_sc as plsc

_BM = 2048
_BN = 2048
_K = 256
_N = 8192


def _vq_argmin_body(z_ref, z2_ref, e_ref, idx_ref, e2_ref, eb_ref):
    i = pl.program_id(0)

    @pl.when(i == 0)
    def _():
        e_all = e_ref[...]
        e2_ref[...] = jnp.sum(e_all * e_all, axis=1, keepdims=True)
        eb_ref[...] = e_all.astype(jnp.bfloat16)

    z = z_ref[...]
    z2t = z2_ref[...]
    zb2 = z.astype(jnp.bfloat16) * jnp.bfloat16(2.0)
    sub8 = lax.broadcasted_iota(jnp.int32, (8, _BM), 0)
    best_val = jnp.full((1, _BM), jnp.inf, dtype=jnp.float32)
    best_idx = jnp.zeros((1, _BM), dtype=jnp.int32)
    for j in range(_N // _BN):
        eb = eb_ref[j * _BN:(j + 1) * _BN, :]
        dot2 = lax.dot_general(eb, zb2, (((1,), (1,)), ((), ())),
                               preferred_element_type=jnp.float32)
        acc_v = jnp.full((8, _BM), jnp.inf, dtype=jnp.float32)
        acc_t = jnp.zeros((8, _BM), dtype=jnp.int32)
        for t in range(_BN // 8):
            r0 = j * _BN + t * 8
            d = (z2t + e2_ref[r0:r0 + 8, :]) - dot2[t * 8:(t + 1) * 8, :]
            m = d < acc_v
            acc_v = jnp.where(m, d, acc_v)
            acc_t = jnp.where(m, jnp.int32(t), acc_t)
        row_min = jnp.min(acc_v, axis=0, keepdims=True)
        n_val = acc_t * 8 + sub8 + j * _BN
        in_idx = jnp.min(jnp.where(acc_v == row_min, n_val, _N),
                         axis=0, keepdims=True)
        upd = row_min < best_val
        best_val = jnp.where(
            upd, row_min.astype(jnp.bfloat16).astype(jnp.float32), best_val)
        best_idx = jnp.where(upd, in_idx, best_idx)
    idx_ref[...] = best_idx


def _argmin_call(z_flat, z2t, embedding):
    b = z_flat.shape[0]
    return pl.pallas_call(
        _vq_argmin_body,
        grid=(b // _BM,),
        in_specs=[
            pl.BlockSpec((_BM, _K), lambda i: (i, 0)),
            pl.BlockSpec((1, _BM), lambda i: (0, i)),
            pl.BlockSpec((_N, _K), lambda i: (0, 0)),
        ],
        out_specs=pl.BlockSpec((1, _BM), lambda i: (0, i)),
        out_shape=jax.ShapeDtypeStruct((1, b), jnp.int32),
        scratch_shapes=[pltpu.VMEM((_N, 1), jnp.float32),
                        pltpu.VMEM((_N, _K), jnp.bfloat16)],
    )(z_flat, z2t, embedding)


def _gather_call(embedding, idx_flat):
    b = idx_flat.shape[0]
    d = embedding.shape[1]
    info = plsc.get_sparse_core_info()
    nc, ns = info.num_cores, info.num_subcores
    nw = nc * ns
    bpw = b // nw

    mesh = plsc.VectorSubcoreMesh(core_axis_name="c", subcore_axis_name="s")

    @functools.partial(
        pl.kernel, mesh=mesh,
        out_type=jax.ShapeDtypeStruct((b, d), jnp.float32),
        scratch_types=[
            pltpu.VMEM((bpw,), jnp.int32),
            pltpu.VMEM((bpw, d), jnp.float32),
            pltpu.SemaphoreType.DMA,
        ],
    )
    def _gather_rows(table_hbm, idx_hbm, out_hbm, idx_v, rows_v, sem):
        wid = lax.axis_index("s") * nc + lax.axis_index("c")
        base = wid * bpw
        pltpu.sync_copy(idx_hbm.at[pl.ds(base, bpw)], idx_v)
        pltpu.async_copy(table_hbm.at[idx_v], rows_v, sem).wait()
        pltpu.sync_copy(rows_v, out_hbm.at[pl.ds(base, bpw)])

    return _gather_rows(embedding, idx_flat)


def kernel(z, embedding, is_training):
    e_dim = embedding.shape[1]
    z_flat = z.reshape(-1, e_dim)
    z2t = jnp.sum(z ** 2, axis=-1).reshape(1, -1)
    idx = _argmin_call(z_flat, z2t, embedding).reshape(-1)
    z_q = _gather_call(embedding, idx).reshape(z.shape)
    return (z_q, idx)
